# baseline (speedup 1.0000x reference)
import functools

import jax
import jax.numpy as jnp
from jax import lax
from jax.experimental import pallas as pl
from jax.experimental.pallas import tpu as pltpu
from jax.experimental.pallas import tpu_sc as plsc

B = 16384
D = 16
L = 16
NC = 2
NS = 16
NW = NC * NS
BPW = B // NW
CHUNK = 128
NCHUNK = BPW // CHUNK


def _mf_body(ui_hbm, ii_hbm, ue_hbm, ie_hbm, out_hbm,
             uidx_v, iidx_v, urows_v, irows_v, prod_v, out_v, usem, isem):
    wid = lax.axis_index("s") * NC + lax.axis_index("c")
    base = wid * BPW

    for j in range(NCHUNK):
        pltpu.sync_copy(ui_hbm.at[pl.ds(base + j * CHUNK, CHUNK)], uidx_v.at[j])
        pltpu.sync_copy(ii_hbm.at[pl.ds(base + j * CHUNK, CHUNK)], iidx_v.at[j])

    copies = []
    for j in range(NCHUNK):
        copies.append(pltpu.async_copy(ue_hbm.at[uidx_v.at[j]], urows_v.at[j], usem))
        copies.append(pltpu.async_copy(ie_hbm.at[iidx_v.at[j]], irows_v.at[j], isem))
    for c in copies:
        c.wait()

    lane16 = lax.iota(jnp.int32, L) * D

    def block(b, _):
        j = b // (CHUNK // L)
        r = (b % (CHUNK // L)) * L
        for k in range(L):
            prod_v[pl.ds(k * D, D)] = urows_v[j, r + k, :] * irows_v[j, r + k, :]
        acc = jnp.zeros((L,), jnp.float32)
        for d in range(D):
            acc = acc + plsc.load_gather(prod_v, [lane16 + d])
        out_v[pl.ds(b * L, L)] = acc
        return 0

    lax.fori_loop(0, BPW // L, block, 0)

    pltpu.sync_copy(out_v, out_hbm.at[pl.ds(base, BPW)])


@functools.partial(
    pl.kernel,
    out_type=jax.ShapeDtypeStruct((B,), jnp.float32),
    mesh=plsc.VectorSubcoreMesh(core_axis_name="c", subcore_axis_name="s"),
    compiler_params=pltpu.CompilerParams(
        needs_layout_passes=False, use_tc_tiling_on_sc=False
    ),
    scratch_types=[
        pltpu.VMEM((NCHUNK, CHUNK), jnp.int32),
        pltpu.VMEM((NCHUNK, CHUNK), jnp.int32),
        pltpu.VMEM((NCHUNK, CHUNK, D), jnp.float32),
        pltpu.VMEM((NCHUNK, CHUNK, D), jnp.float32),
        pltpu.VMEM((L * D,), jnp.float32),
        pltpu.VMEM((BPW,), jnp.float32),
        pltpu.SemaphoreType.DMA,
        pltpu.SemaphoreType.DMA,
    ],
)
def _mf_kernel(*refs):
    _mf_body(*refs)


def kernel(user_indices, item_indices, user_embedding, item_embedding):
    return _mf_kernel(user_indices, item_indices, user_embedding, item_embedding)


# speedup vs baseline: 5.4307x; 5.4307x over previous
import functools

import jax
import jax.numpy as jnp
from jax import lax
from jax.experimental import pallas as pl
from jax.experimental.pallas import tpu as pltpu
from jax.experimental.pallas import tpu_sc as plsc

B = 16384
D = 16
L = 16
NC = 2
NS = 16
NW = NC * NS
BPW = B // NW
TL = 128
RING = L * TL


def _mf_body(ui_hbm, ii_hbm, ue_hbm, ie_hbm, out_hbm,
             uidx_v, iidx_v, uring, iring, prod_v, out_v, usem, isem):
    wid = lax.axis_index("s") * NC + lax.axis_index("c")
    base = wid * BPW

    pltpu.sync_copy(ui_hbm.at[pl.ds(base, BPW)], uidx_v)
    pltpu.sync_copy(ii_hbm.at[pl.ds(base, BPW)], iidx_v)

    lane = lax.iota(jnp.int32, L)
    lane16 = lane * D

    def block(p, _):
        s = p * L
        uvec = uidx_v[pl.ds(s, L)]
        ivec = iidx_v[pl.ds(s, L)]
        for j in range(L):
            uc = pl.multiple_of((uvec[j] // TL) * TL, TL)
            ic = pl.multiple_of((ivec[j] // TL) * TL, TL)
            pltpu.async_copy(ue_hbm.at[:, pl.ds(uc, TL)],
                             uring.at[:, pl.ds(j * TL, TL)], usem)
            pltpu.async_copy(ie_hbm.at[:, pl.ds(ic, TL)],
                             iring.at[:, pl.ds(j * TL, TL)], isem)
        pltpu.make_async_copy(ue_hbm.at[:, pl.ds(0, RING)], uring, usem).wait()
        pltpu.make_async_copy(ie_hbm.at[:, pl.ds(0, RING)], iring, isem).wait()
        for j in range(L):
            ucc = jnp.full((L,), j * TL, jnp.int32) + uvec[j] % TL
            icc = jnp.full((L,), j * TL, jnp.int32) + ivec[j] % TL
            ucol = plsc.load_gather(uring, [lane, ucc])
            icol = plsc.load_gather(iring, [lane, icc])
            prod_v[pl.ds(j * D, D)] = ucol * icol
        acc = plsc.load_gather(prod_v, [lane16])
        for d in range(1, D):
            acc = acc + plsc.load_gather(prod_v, [lane16 + d])
        out_v[pl.ds(s, L)] = acc
        return 0

    lax.fori_loop(0, BPW // L, block, 0)

    pltpu.sync_copy(out_v, out_hbm.at[pl.ds(base, BPW)])


@functools.partial(
    pl.kernel,
    out_type=jax.ShapeDtypeStruct((B,), jnp.float32),
    mesh=plsc.VectorSubcoreMesh(core_axis_name="c", subcore_axis_name="s"),
    compiler_params=pltpu.CompilerParams(
        needs_layout_passes=False, use_tc_tiling_on_sc=True
    ),
    scratch_types=[
        pltpu.VMEM((BPW,), jnp.int32),
        pltpu.VMEM((BPW,), jnp.int32),
        pltpu.VMEM((D, RING), jnp.float32),
        pltpu.VMEM((D, RING), jnp.float32),
        pltpu.VMEM((L * D,), jnp.float32),
        pltpu.VMEM((BPW,), jnp.float32),
        pltpu.SemaphoreType.DMA,
        pltpu.SemaphoreType.DMA,
    ],
)
def _mf_kernel(*refs):
    _mf_body(*refs)


def kernel(user_indices, item_indices, user_embedding, item_embedding):
    return _mf_kernel(user_indices, item_indices,
                      user_embedding.T, item_embedding.T)


# speedup vs baseline: 5.4922x; 1.0113x over previous
import functools

import jax
import jax.numpy as jnp
from jax import lax
from jax.experimental import pallas as pl
from jax.experimental.pallas import tpu as pltpu
from jax.experimental.pallas import tpu_sc as plsc

B = 16384
D = 16
L = 16
NC = 2
NS = 16
NW = NC * NS
BPW = B // NW
TL = 128
G = 8
RING = G * TL


def _mf_body(ui_hbm, ii_hbm, ue_hbm, ie_hbm, out_hbm,
             uidx_v, iidx_v, ua, ub, ia, ib, prod_v, out_v,
             uasem, ubsem, iasem, ibsem):
    wid = lax.axis_index("s") * NC + lax.axis_index("c")
    base = wid * BPW

    pltpu.sync_copy(ui_hbm.at[pl.ds(base, BPW)], uidx_v)
    pltpu.sync_copy(ii_hbm.at[pl.ds(base, BPW)], iidx_v)

    lane = lax.iota(jnp.int32, L)
    lane16 = lane * D

    def issue_half(uvec, ivec, lo, uring, iring, us, isem_):
        for j in range(G):
            uc = pl.multiple_of((uvec[lo + j] // TL) * TL, TL)
            ic = pl.multiple_of((ivec[lo + j] // TL) * TL, TL)
            pltpu.async_copy(ue_hbm.at[:, pl.ds(uc, TL)],
                             uring.at[:, pl.ds(j * TL, TL)], us)
            pltpu.async_copy(ie_hbm.at[:, pl.ds(ic, TL)],
                             iring.at[:, pl.ds(j * TL, TL)], isem_)

    def drain_half(uring, iring, us, isem_):
        pltpu.make_async_copy(ue_hbm.at[:, pl.ds(0, RING)], uring, us).wait()
        pltpu.make_async_copy(ie_hbm.at[:, pl.ds(0, RING)], iring, isem_).wait()

    def process_half(uvec, ivec, lo, uring, iring, pbase):
        for j in range(G):
            ucc = jnp.full((L,), j * TL, jnp.int32) + uvec[lo + j] % TL
            icc = jnp.full((L,), j * TL, jnp.int32) + ivec[lo + j] % TL
            ucol = plsc.load_gather(uring, [lane, ucc])
            icol = plsc.load_gather(iring, [lane, icc])
            prod_v[pl.ds(pbase + j * D, D)] = ucol * icol

    def block(p, _):
        s = p * L
        uvec = uidx_v[pl.ds(s, L)]
        ivec = iidx_v[pl.ds(s, L)]
        issue_half(uvec, ivec, 0, ua, ia, uasem, iasem)
        issue_half(uvec, ivec, G, ub, ib, ubsem, ibsem)
        drain_half(ua, ia, uasem, iasem)
        process_half(uvec, ivec, 0, ua, ia, 0)
        drain_half(ub, ib, ubsem, ibsem)
        process_half(uvec, ivec, G, ub, ib, G * D)
        acc = plsc.load_gather(prod_v, [lane16])
        for d in range(1, D):
            acc = acc + plsc.load_gather(prod_v, [lane16 + d])
        out_v[pl.ds(s, L)] = acc
        return 0

    lax.fori_loop(0, BPW // L, block, 0)

    pltpu.sync_copy(out_v, out_hbm.at[pl.ds(base, BPW)])


@functools.partial(
    pl.kernel,
    out_type=jax.ShapeDtypeStruct((B,), jnp.float32),
    mesh=plsc.VectorSubcoreMesh(core_axis_name="c", subcore_axis_name="s"),
    compiler_params=pltpu.CompilerParams(
        needs_layout_passes=False, use_tc_tiling_on_sc=True
    ),
    scratch_types=[
        pltpu.VMEM((BPW,), jnp.int32),
        pltpu.VMEM((BPW,), jnp.int32),
        pltpu.VMEM((D, RING), jnp.float32),
        pltpu.VMEM((D, RING), jnp.float32),
        pltpu.VMEM((D, RING), jnp.float32),
        pltpu.VMEM((D, RING), jnp.float32),
        pltpu.VMEM((L * D,), jnp.float32),
        pltpu.VMEM((BPW,), jnp.float32),
        pltpu.SemaphoreType.DMA,
        pltpu.SemaphoreType.DMA,
        pltpu.SemaphoreType.DMA,
        pltpu.SemaphoreType.DMA,
    ],
)
def _mf_kernel(*refs):
    _mf_body(*refs)


def kernel(user_indices, item_indices, user_embedding, item_embedding):
    return _mf_kernel(user_indices, item_indices,
                      user_embedding.T, item_embedding.T)
